# baseline (speedup 1.0000x reference)
import functools
import math

import jax
import jax.numpy as jnp
from jax import lax
from jax.experimental import pallas as pl
from jax.experimental.pallas import tpu as pltpu
from jax.experimental.pallas import tpu_sc as plsc

V = 4096
D = 2048
BATCH = 4
B_TOTAL = BATCH * 4096
FMT_EPS = 0.1
INP_EPS = 0.1
MIN_VALUE = 1e-10


_ROWS_BLK = 512


def _combine_body(mask_ref, emb_ref, delta_ref, out_ref):
    delta = delta_ref[...]
    norm = jnp.sqrt(jnp.sum(delta * delta, axis=1, keepdims=True))
    m = mask_ref[...]
    fmt = jnp.float32(FMT_EPS * math.sqrt(D / 2048.0))
    inp = jnp.float32(INP_EPS * math.sqrt(D / 2048.0))
    eps = jnp.full(m.shape, MIN_VALUE, dtype=jnp.float32)
    pos = m > 0
    r = jnp.remainder(m, 4)
    eps = jnp.where(pos & (r == 1), fmt, eps)
    eps = jnp.where(pos & (r == 3), fmt, eps)
    eps = jnp.where(pos & (r == 2), inp, eps)
    scale = eps / jnp.maximum(norm, eps)
    out_ref[...] = emb_ref[...] + delta * scale


def _combined_table(embedding, delta, mask2d):
    return pl.pallas_call(
        _combine_body,
        grid=(V // _ROWS_BLK,),
        in_specs=[
            pl.BlockSpec((_ROWS_BLK, 1), lambda i: (i, 0)),
            pl.BlockSpec((_ROWS_BLK, D), lambda i: (i, 0)),
            pl.BlockSpec((_ROWS_BLK, D), lambda i: (i, 0)),
        ],
        out_specs=pl.BlockSpec((_ROWS_BLK, D), lambda i: (i, 0)),
        out_shape=jax.ShapeDtypeStruct((V, D), jnp.float32),
    )(mask2d, embedding, delta)


_NC = 2
_NS = 16
_NW = _NC * _NS
_BPW = B_TOTAL // _NW
_K = 8
_NCH = _BPW // _K
_NB = 6
_LEAD = _NB - 2
_TAIL0 = ((_NCH - _LEAD) // _NB) * _NB
assert _TAIL0 >= _NB and _TAIL0 % _NB == 0


_WPB = 4096 // _BPW


def _gather_body(table, idx, out, idx_v, *rest):
    bufs = rest[:_NB]
    gsems = rest[_NB:2 * _NB]
    ssems = rest[2 * _NB:]
    wid = lax.axis_index("s") * _NC + lax.axis_index("c")
    batch = wid // _WPB
    col0 = (wid % _WPB) * _BPW
    pltpu.sync_copy(idx.at[pl.ds(wid * _BPW, _BPW)], idx_v)

    def gather(j, b):
        pltpu.async_copy(table.at[idx_v.at[pl.ds(j * _K, _K)]], bufs[b], gsems[b])

    def wait_gather(j, b):
        pltpu.make_async_copy(
            table.at[idx_v.at[pl.ds(j * _K, _K)]], bufs[b], gsems[b]
        ).wait()

    def scatter(j, b):
        pltpu.async_copy(
            bufs[b], out.at[batch, pl.ds(col0 + j * _K, _K)], ssems[b]
        )

    def wait_scatter(j, b):
        pltpu.make_async_copy(
            bufs[b], out.at[batch, pl.ds(col0 + j * _K, _K)], ssems[b]
        ).wait()

    def position(j, b, do_wait, do_gather):
        if do_wait:
            wait_scatter(j - 2, (b - 2) % _NB)
        if do_gather:
            gather(j + _LEAD, (b + _LEAD) % _NB)
        wait_gather(j, b)
        scatter(j, b)

    for j in range(_LEAD):
        gather(j, j % _NB)
    for b in range(_NB):
        position(b, b, b >= 2, True)

    def rev(r, carry):
        j0 = r * _NB
        for b in range(_NB):
            position(j0 + b, b, True, True)
        return carry

    lax.fori_loop(1, _TAIL0 // _NB, rev, 0)

    for j in range(_TAIL0, _NCH):
        position(j, j % _NB, True, j + _LEAD < _NCH)
    wait_scatter(_NCH - 2, (_NCH - 2) % _NB)
    wait_scatter(_NCH - 1, (_NCH - 1) % _NB)


@functools.lru_cache(maxsize=1)
def _make_gather_kernel():
    mesh = plsc.VectorSubcoreMesh(core_axis_name="c", subcore_axis_name="s")
    return pl.kernel(
        _gather_body,
        out_type=jax.ShapeDtypeStruct((BATCH, 4096, D), jnp.float32),
        mesh=mesh,
        scratch_types=(
            [pltpu.VMEM((_BPW,), jnp.int32)]
            + [pltpu.VMEM((_K, D), jnp.float32)] * _NB
            + [pltpu.SemaphoreType.DMA] * (2 * _NB)
        ),
    )


def kernel(indices, embedding, delta_embedding, cpt_tokens_type_mask):
    mask2d = cpt_tokens_type_mask.reshape(V, 1)
    combined = _combined_table(embedding, delta_embedding, mask2d)
    return _make_gather_kernel()(combined, indices.astype(jnp.int32).reshape(-1))


# speedup vs baseline: 1.0147x; 1.0147x over previous
import functools
import math

import jax
import jax.numpy as jnp
from jax import lax
from jax.experimental import pallas as pl
from jax.experimental.pallas import tpu as pltpu
from jax.experimental.pallas import tpu_sc as plsc

V = 4096
D = 2048
BATCH = 4
B_TOTAL = BATCH * 4096
FMT_EPS = 0.1
INP_EPS = 0.1
MIN_VALUE = 1e-10


_ROWS_BLK = 512


def _combine_body(mask_ref, emb_ref, delta_ref, out_ref):
    delta = delta_ref[...]
    norm = jnp.sqrt(jnp.sum(delta * delta, axis=1, keepdims=True))
    m = mask_ref[...]
    fmt = jnp.float32(FMT_EPS * math.sqrt(D / 2048.0))
    inp = jnp.float32(INP_EPS * math.sqrt(D / 2048.0))
    eps = jnp.full(m.shape, MIN_VALUE, dtype=jnp.float32)
    pos = m > 0
    r = jnp.remainder(m, 4)
    eps = jnp.where(pos & (r == 1), fmt, eps)
    eps = jnp.where(pos & (r == 3), fmt, eps)
    eps = jnp.where(pos & (r == 2), inp, eps)
    scale = eps / jnp.maximum(norm, eps)
    out_ref[...] = emb_ref[...] + delta * scale


def _combined_table(embedding, delta, mask2d):
    return pl.pallas_call(
        _combine_body,
        grid=(V // _ROWS_BLK,),
        in_specs=[
            pl.BlockSpec((_ROWS_BLK, 1), lambda i: (i, 0)),
            pl.BlockSpec((_ROWS_BLK, D), lambda i: (i, 0)),
            pl.BlockSpec((_ROWS_BLK, D), lambda i: (i, 0)),
        ],
        out_specs=pl.BlockSpec((_ROWS_BLK, D), lambda i: (i, 0)),
        out_shape=jax.ShapeDtypeStruct((V, D), jnp.float32),
    )(mask2d, embedding, delta)


_NC = 2
_NS = 16
_NW = _NC * _NS
_BPW = B_TOTAL // _NW
_K = 8
_NCH = _BPW // _K
_NB = 6
_LEAD = _NB - 2
_TAIL0 = ((_NCH - _LEAD) // _NB) * _NB
assert _TAIL0 >= _NB and _TAIL0 % _NB == 0


_WPB = 4096 // _BPW


def _gather_body(table, idx, out, idx_v, *rest):
    bufs = rest[:_NB]
    gsems = rest[_NB:2 * _NB]
    ssems = rest[2 * _NB:]
    wid = lax.axis_index("s") * _NC + lax.axis_index("c")
    batch = wid // _WPB
    col0 = (wid % _WPB) * _BPW
    pltpu.sync_copy(idx.at[batch, pl.ds(col0, _BPW)], idx_v)

    def gather(j, b):
        pltpu.async_copy(table.at[idx_v.at[pl.ds(j * _K, _K)]], bufs[b], gsems[b])

    def wait_gather(j, b):
        pltpu.make_async_copy(
            table.at[idx_v.at[pl.ds(j * _K, _K)]], bufs[b], gsems[b]
        ).wait()

    def scatter(j, b):
        pltpu.async_copy(
            bufs[b], out.at[batch, pl.ds(col0 + j * _K, _K)], ssems[b]
        )

    def wait_scatter(j, b):
        pltpu.make_async_copy(
            bufs[b], out.at[batch, pl.ds(col0 + j * _K, _K)], ssems[b]
        ).wait()

    def position(j, b, do_wait, do_gather):
        if do_wait:
            wait_scatter(j - 2, (b - 2) % _NB)
        if do_gather:
            gather(j + _LEAD, (b + _LEAD) % _NB)
        wait_gather(j, b)
        scatter(j, b)

    for j in range(_LEAD):
        gather(j, j % _NB)
    for b in range(_NB):
        position(b, b, b >= 2, True)

    def rev(r, carry):
        j0 = r * _NB
        for b in range(_NB):
            position(j0 + b, b, True, True)
        return carry

    lax.fori_loop(1, _TAIL0 // _NB, rev, 0)

    for j in range(_TAIL0, _NCH):
        position(j, j % _NB, True, j + _LEAD < _NCH)
    wait_scatter(_NCH - 2, (_NCH - 2) % _NB)
    wait_scatter(_NCH - 1, (_NCH - 1) % _NB)


@functools.lru_cache(maxsize=1)
def _make_gather_kernel():
    mesh = plsc.VectorSubcoreMesh(core_axis_name="c", subcore_axis_name="s")
    return pl.kernel(
        _gather_body,
        out_type=jax.ShapeDtypeStruct((BATCH, 4096, D), jnp.float32),
        mesh=mesh,
        scratch_types=(
            [pltpu.VMEM((_BPW,), jnp.int32)]
            + [pltpu.VMEM((_K, D), jnp.float32)] * _NB
            + [pltpu.SemaphoreType.DMA] * (2 * _NB)
        ),
    )


def kernel(indices, embedding, delta_embedding, cpt_tokens_type_mask):
    mask2d = cpt_tokens_type_mask.reshape(V, 1)
    combined = _combined_table(embedding, delta_embedding, mask2d)
    return _make_gather_kernel()(combined, indices.astype(jnp.int32))


# speedup vs baseline: 1.0182x; 1.0035x over previous
import functools
import math

import jax
import jax.numpy as jnp
from jax import lax
from jax.experimental import pallas as pl
from jax.experimental.pallas import tpu as pltpu
from jax.experimental.pallas import tpu_sc as plsc

V = 4096
D = 2048
BATCH = 4
B_TOTAL = BATCH * 4096
FMT_EPS = 0.1
INP_EPS = 0.1
MIN_VALUE = 1e-10


_ROWS_BLK = 512


def _combine_body(mask_ref, emb_ref, delta_ref, out_ref):
    delta = delta_ref[...]
    norm = jnp.sqrt(jnp.sum(delta * delta, axis=1, keepdims=True))
    m = mask_ref[...]
    fmt = jnp.float32(FMT_EPS * math.sqrt(D / 2048.0))
    inp = jnp.float32(INP_EPS * math.sqrt(D / 2048.0))
    eps = jnp.full(m.shape, MIN_VALUE, dtype=jnp.float32)
    pos = m > 0
    r = jnp.remainder(m, 4)
    eps = jnp.where(pos & (r == 1), fmt, eps)
    eps = jnp.where(pos & (r == 3), fmt, eps)
    eps = jnp.where(pos & (r == 2), inp, eps)
    scale = eps / jnp.maximum(norm, eps)
    out_ref[...] = emb_ref[...] + delta * scale


def _combined_table(embedding, delta, mask2d):
    return pl.pallas_call(
        _combine_body,
        grid=(V // _ROWS_BLK,),
        in_specs=[
            pl.BlockSpec((_ROWS_BLK, 1), lambda i: (i, 0)),
            pl.BlockSpec((_ROWS_BLK, D), lambda i: (i, 0)),
            pl.BlockSpec((_ROWS_BLK, D), lambda i: (i, 0)),
        ],
        out_specs=pl.BlockSpec((_ROWS_BLK, D), lambda i: (i, 0)),
        out_shape=jax.ShapeDtypeStruct((V, D), jnp.float32),
    )(mask2d, embedding, delta)


_NC = 2
_NS = 16
_NW = _NC * _NS
_BPW = B_TOTAL // _NW
_K = 8
_NCH = _BPW // _K
_NB = 7
_LEAD = _NB - 2
_TAIL0 = ((_NCH - _LEAD) // _NB) * _NB
assert _TAIL0 >= _NB and _TAIL0 % _NB == 0


_WPB = 4096 // _BPW


def _gather_body(table, idx, out, idx_v, *rest):
    bufs = rest[:_NB]
    gsems = rest[_NB:2 * _NB]
    ssems = rest[2 * _NB:]
    wid = lax.axis_index("s") * _NC + lax.axis_index("c")
    batch = wid // _WPB
    col0 = (wid % _WPB) * _BPW
    pltpu.sync_copy(idx.at[batch, pl.ds(col0, _BPW)], idx_v)

    def gather(j, b):
        pltpu.async_copy(table.at[idx_v.at[pl.ds(j * _K, _K)]], bufs[b], gsems[b])

    def wait_gather(j, b):
        pltpu.make_async_copy(
            table.at[idx_v.at[pl.ds(j * _K, _K)]], bufs[b], gsems[b]
        ).wait()

    def scatter(j, b):
        pltpu.async_copy(
            bufs[b], out.at[batch, pl.ds(col0 + j * _K, _K)], ssems[b]
        )

    def wait_scatter(j, b):
        pltpu.make_async_copy(
            bufs[b], out.at[batch, pl.ds(col0 + j * _K, _K)], ssems[b]
        ).wait()

    def position(j, b, do_wait, do_gather):
        if do_wait:
            wait_scatter(j - 2, (b - 2) % _NB)
        if do_gather:
            gather(j + _LEAD, (b + _LEAD) % _NB)
        wait_gather(j, b)
        scatter(j, b)

    for j in range(_LEAD):
        gather(j, j % _NB)
    for b in range(_NB):
        position(b, b, b >= 2, True)

    def rev(r, carry):
        j0 = r * _NB
        for b in range(_NB):
            position(j0 + b, b, True, True)
        return carry

    lax.fori_loop(1, _TAIL0 // _NB, rev, 0)

    for j in range(_TAIL0, _NCH):
        position(j, j % _NB, True, j + _LEAD < _NCH)
    wait_scatter(_NCH - 2, (_NCH - 2) % _NB)
    wait_scatter(_NCH - 1, (_NCH - 1) % _NB)


@functools.lru_cache(maxsize=1)
def _make_gather_kernel():
    mesh = plsc.VectorSubcoreMesh(core_axis_name="c", subcore_axis_name="s")
    return pl.kernel(
        _gather_body,
        out_type=jax.ShapeDtypeStruct((BATCH, 4096, D), jnp.float32),
        mesh=mesh,
        scratch_types=(
            [pltpu.VMEM((_BPW,), jnp.int32)]
            + [pltpu.VMEM((_K, D), jnp.float32)] * _NB
            + [pltpu.SemaphoreType.DMA] * (2 * _NB)
        ),
    )


def kernel(indices, embedding, delta_embedding, cpt_tokens_type_mask):
    mask2d = cpt_tokens_type_mask.reshape(V, 1)
    combined = _combined_table(embedding, delta_embedding, mask2d)
    return _make_gather_kernel()(combined, indices.astype(jnp.int32))
